# baseline (speedup 1.0000x reference)
import functools

import jax
import jax.numpy as jnp
from jax import lax
from jax.experimental import pallas as pl
from jax.experimental.pallas import tpu as pltpu
from jax.experimental.pallas import tpu_sc as plsc

_D = 32
_B = 16384
_NC = 2
_NS = 16
_NW = _NC * _NS
_BPW = _B // _NW
_CHUNK = 128
_NCHUNK = _BPW // _CHUNK
_GROUP = 16
_NGROUP = _BPW // _GROUP

_mesh = plsc.VectorSubcoreMesh(core_axis_name="c", subcore_axis_name="s")


@functools.partial(
    pl.kernel,
    mesh=_mesh,
    out_type=jax.ShapeDtypeStruct((_B,), jnp.float32),
    compiler_params=pltpu.CompilerParams(
        needs_layout_passes=False, use_tc_tiling_on_sc=False),
    scratch_types=[
        pltpu.VMEM((_NCHUNK, _CHUNK), jnp.int32),
        pltpu.VMEM((_NCHUNK, _CHUNK), jnp.int32),
        pltpu.VMEM((_BPW, _D), jnp.float32),
        pltpu.VMEM((_BPW, _D), jnp.float32),
        pltpu.VMEM((_D,), jnp.float32),
        pltpu.VMEM((16,), jnp.float32),
        pltpu.VMEM((_BPW,), jnp.float32),
        pltpu.VMEM((_GROUP, 16), jnp.float32),
        pltpu.SemaphoreType.DMA,
    ],
)
def _gmf_sc(uidx_hbm, iidx_hbm, utab_hbm, itab_hbm, w_hbm, b_hbm, out_hbm,
            uidx_v, iidx_v, urows_v, irows_v, w_v, b_v, out_v, sred_v, sem):
    cid = lax.axis_index("c")
    sid = lax.axis_index("s")
    wid = sid * _NC + cid
    base = wid * _BPW

    pltpu.sync_copy(uidx_hbm.at[wid], uidx_v)
    pltpu.sync_copy(iidx_hbm.at[wid], iidx_v)
    pltpu.sync_copy(w_hbm, w_v)
    pltpu.sync_copy(b_hbm, b_v)

    copies = []
    for j in range(_NCHUNK):
        copies.append(pltpu.async_copy(
            utab_hbm.at[uidx_v.at[j]],
            urows_v.at[pl.ds(j * _CHUNK, _CHUNK)], sem))
        copies.append(pltpu.async_copy(
            itab_hbm.at[iidx_v.at[j]],
            irows_v.at[pl.ds(j * _CHUNK, _CHUNK)], sem))
    for c in copies:
        c.wait()

    w0 = w_v[pl.ds(0, 16)]
    w1 = w_v[pl.ds(16, 16)]
    bias = b_v[...]
    lane = lax.iota(jnp.int32, 16)
    col_idx = [jnp.full((16,), c, jnp.int32) for c in range(16)]

    def group_body(g, carry):
        rbase = g * _GROUP
        for r in range(_GROUP):
            row = rbase + r
            u0 = urows_v[row, pl.ds(0, 16)]
            u1 = urows_v[row, pl.ds(16, 16)]
            i0 = irows_v[row, pl.ds(0, 16)]
            i1 = irows_v[row, pl.ds(16, 16)]
            sred_v[r, pl.ds(0, 16)] = u0 * i0 * w0 + u1 * i1 * w1
        acc = bias
        for c in range(16):
            acc = acc + plsc.load_gather(sred_v, [lane, col_idx[c]])
        out_v[pl.ds(rbase, 16)] = 1.0 / (1.0 + jnp.exp(-acc))
        return carry

    lax.fori_loop(0, _NGROUP, group_body, 0)

    pltpu.sync_copy(out_v, out_hbm.at[pl.ds(base, _BPW)])


def kernel(user_indices, item_indices, user_table, item_table, W, b):
    uidx = user_indices.reshape(_NW, _NCHUNK, _CHUNK).astype(jnp.int32)
    iidx = item_indices.reshape(_NW, _NCHUNK, _CHUNK).astype(jnp.int32)
    wvec = W.reshape(_D)
    bvec = jnp.broadcast_to(b.reshape(1), (16,))
    return _gmf_sc(uidx, iidx, user_table, item_table, wvec, bvec)
